# baseline (speedup 1.0000x reference)
import functools

import numpy as np

import jax
import jax.numpy as jnp
from jax.experimental import pallas as pl
from jax.experimental.pallas import tpu as pltpu

_C = 256
_NC = 80
_SC = 4
_G = 32
_STRIDES = (8, 16, 32, 64, 128)


def _dot(a, b):
    return jax.lax.dot_general(a, b, (((1,), (0,)), ((), ())),
                               preferred_element_type=jnp.float32)


def _head_fn(x_ref, cw_ref, cb_ref, cg_ref, cbe_ref,
             rw_ref, rb_ref, rg_ref, rbe_ref,
             pcw_ref, pcb_ref, prw_ref, prb_ref,
             m_ref, u_ref, gm_ref,
             o_ref,
             t0c, t0p, t0m, t1c, t1p, t1m, *, cfg):
    Wc, P, LO, M = cfg["Wc"], cfg["P"], cfg["LO"], cfg["M"]
    ranges, segs, zeros = cfg["ranges"], cfg["segs"], cfg["zeros"]
    offs = [dy * Wc + dx for dy in (-1, 0, 1) for dx in (-1, 0, 1)]
    gmat = gm_ref[...]
    T0 = (t0c, t0p, t0m)
    T1 = (t1c, t1p, t1m)

    def load_x(dst3):
        c, p, m = dst3
        c[...] = x_ref[...]
        p[pl.ds(0, P - 1), :] = x_ref[pl.ds(1, P - 1), :]
        p[pl.ds(P - 1, 1), :] = jnp.zeros((1, _C), jnp.bfloat16)
        m[pl.ds(1, P - 1), :] = x_ref[pl.ds(0, P - 1), :]
        m[pl.ds(0, 1), :] = jnp.zeros((1, _C), jnp.bfloat16)

    def conv9(src3, w_ref, widx):
        acc = None
        for t in range(9):
            dy, dx = t // 3 - 1, t % 3 - 1
            buf = src3[0] if dx == 0 else (src3[1] if dx == 1 else src3[2])
            a = buf[pl.ds(LO + dy * Wc, M), :]
            d = _dot(a, w_ref[widx, t])
            acc = d if acc is None else acc + d
        return acc

    def store_triple(dst3, r0, n, zb):
        dst3[0][pl.ds(r0, n), :] = zb
        dst3[1][pl.ds(r0 - 1, n), :] = zb
        dst3[2][pl.ds(r0 + 1, n), :] = zb

    def zero_triple(dst3, z0, z1):
        dst3[0][pl.ds(z0, z1 - z0), :] = jnp.zeros((z1 - z0, _C), jnp.bfloat16)
        p0, p1 = max(z0 - 1, 0), max(z1 - 1, 0)
        if p1 > p0:
            dst3[1][pl.ds(p0, p1 - p0), :] = jnp.zeros((p1 - p0, _C),
                                                       jnp.bfloat16)
        m0, m1 = min(z0 + 1, P), min(z1 + 1, P)
        if m1 > m0:
            dst3[2][pl.ds(m0, m1 - m0), :] = jnp.zeros((m1 - m0, _C),
                                                       jnp.bfloat16)

    def gn_relu(y, g_ref, be_ref, i, dst3):
        stats = []
        for (r0, r1, mc, cnt) in ranges:
            ys = y[r0 - LO:r1 - LO]
            mm = m_ref[pl.ds(r0, r1 - r0), mc:mc + 1]
            ym = ys * mm
            stats.append(jnp.sum(ym, 0, keepdims=True))
            stats.append(jnp.sum(ym * ym, 0, keepdims=True))
        st = jnp.concatenate(stats, 0)
        hi = st.astype(jnp.bfloat16)
        lo_ = (st - hi.astype(jnp.float32)).astype(jnp.bfloat16)
        gs = _dot(hi, gmat) + _dot(lo_, gmat)
        g = g_ref[i:i + 1]
        be = be_ref[i:i + 1]
        scs, shs = [], []
        for r, (r0, r1, mc, cnt) in enumerate(ranges):
            mean = gs[2 * r:2 * r + 1] / cnt
            ex2 = gs[2 * r + 1:2 * r + 2] / cnt
            inv = jax.lax.rsqrt(ex2 - mean * mean + 1e-5)
            sc = g * inv
            scs.append(sc)
            shs.append(be - mean * sc)
        for (r0, r1, comps) in segs:
            ys = y[r0 - LO:r1 - LO]
            z = None
            for (mc, ridx) in comps:
                mm = m_ref[pl.ds(r0, r1 - r0), mc:mc + 1]
                zz = jnp.maximum(ys * scs[ridx] + shs[ridx], 0.0) * mm
                z = zz if z is None else z + zz
            store_triple(dst3, r0, r1 - r0, z.astype(jnp.bfloat16))
        for (z0, z1) in zeros:
            zero_triple(dst3, z0, z1)
        dst3[1][pl.ds(P - 1, 1), :] = jnp.zeros((1, _C), jnp.bfloat16)
        dst3[2][pl.ds(0, 1), :] = jnp.zeros((1, _C), jnp.bfloat16)

    def branch(first3, w_ref, g_ref, be_ref):
        load_x(first3)
        trips = [first3, T1 if first3 is T0 else T0]
        src = first3
        for i in range(_SC):
            dst3 = trips[(i + 1) % 2]
            y = conv9(src, w_ref, i)
            gn_relu(y, g_ref, be_ref, i, dst3)
            src = dst3
        return src

    src = branch(T0, cw_ref, cg_ref, cbe_ref)
    yc = pcb_ref[0:1]
    for kx in range(3):
        buf = (src[2], src[0], src[1])[kx]
        ptc = _dot(buf[...], pcw_ref[:, kx * 240:(kx + 1) * 240])
        for ky in range(3):
            r0 = LO + (ky - 1) * Wc
            yc = yc + ptc[r0:r0 + M, ky * _NC:(ky + 1) * _NC]

    src = branch(T1, rw_ref, rg_ref, rbe_ref)
    yr = prb_ref[0:1]
    for kx in range(3):
        buf = (src[2], src[0], src[1])[kx]
        prc = _dot(buf[...], prw_ref[:, kx * 12:(kx + 1) * 12])
        for ky in range(3):
            r0 = LO + (ky - 1) * Wc
            yr = yr + prc[r0:r0 + M, ky * 4:(ky + 1) * 4]
    parts = []
    for (r0, r1, comps) in segs:
        ys = yr[r0 - LO:r1 - LO]
        z = None
        for (mc, ridx) in comps:
            mm = m_ref[pl.ds(r0, r1 - r0), mc:mc + 1]
            uu = u_ref[0:1, ridx:ridx + 1]
            zz = jnp.maximum(ys * uu, 0.0) * mm
            z = zz if z is None else z + zz
        parts.append(z)
    o_ref[pl.ds(LO, M), :] = jnp.concatenate(
        [yc, jnp.concatenate(parts, axis=0)], axis=1)
    o_ref[pl.ds(0, LO), :] = jnp.zeros((LO, _NC + 4), jnp.float32)
    o_ref[pl.ds(P - LO, LO), :] = jnp.zeros((LO, _NC + 4), jnp.float32)


def _np_mask(rows, Wc, rects):
    m = np.zeros((rows, Wc, len(rects)), np.float32)
    for i, (r0, r1, c0, c1) in enumerate(rects):
        m[r0:r1, c0:c1, i] = 1.0
    return m.reshape(rows * Wc, len(rects))


_CFG_A = dict(Wc=72, P=4752, LO=72, M=4608,
              ranges=[(72, 4680, 0, 8 * 4096)],
              segs=[(72, 4680, [(0, 0)])],
              zeros=[(0, 72), (4680, 4752)])

_CFG_B = dict(Wc=40, P=2080, LO=40, M=2000,
              ranges=[(40, 1360, 0, 8 * 1024),
                      (1360, 2040, 1, 8 * 256),
                      (1360, 1760, 2, 8 * 64),
                      (1360, 1600, 3, 8 * 16)],
              segs=[(40, 1360, [(0, 0)]),
                    (1360, 2040, [(1, 1), (2, 2), (3, 3)])],
              zeros=[(0, 40), (2040, 2080)])

_MASK_A = _np_mask(66, 72, [(1, 65, 1, 65)])
_MASK_B = _np_mask(52, 40, [(1, 33, 1, 33),
                            (35, 51, 1, 17),
                            (35, 43, 19, 27),
                            (35, 39, 29, 33)])

_GMAT = (np.arange(_C)[:, None] // 8 == np.arange(_C)[None, :] // 8
         ).astype(np.float32)


def _run_canvas(x, cfg, marr, usml, cw, cb, cg, cbe, rw, rb, rg, rbe,
                pcw, pcb2, prw, prb2):
    B = x.shape[0]
    P = cfg["P"]
    ncol = marr.shape[1]
    gmat = jnp.asarray(_GMAT, jnp.bfloat16)
    kfn = functools.partial(_head_fn, cfg=cfg)
    full = lambda *shape: pl.BlockSpec(shape, lambda b: (0,) * len(shape))
    act = lambda: pltpu.VMEM((P, _C), jnp.bfloat16)
    out = pl.pallas_call(
        kfn,
        grid=(B,),
        in_specs=[
            pl.BlockSpec((None, P, _C), lambda b: (b, 0, 0)),
            full(_SC, 9, _C, _C), full(_SC, _C), full(_SC, _C), full(_SC, _C),
            full(_SC, 9, _C, _C), full(_SC, _C), full(_SC, _C), full(_SC, _C),
            full(_C, 9 * _NC), full(1, _NC), full(_C, 36), full(1, 4),
            full(P, ncol), full(1, 8), full(_C, _C),
        ],
        out_specs=pl.BlockSpec((None, P, _NC + 4), lambda b: (b, 0, 0)),
        out_shape=jax.ShapeDtypeStruct((B, P, _NC + 4), jnp.float32),
        scratch_shapes=[act(), act(), act(), act(), act(), act()],
        compiler_params=pltpu.CompilerParams(
            dimension_semantics=("parallel",)),
    )(x, cw, cb, cg, cbe, rw, rb, rg, rbe, pcw, pcb2, prw, prb2, marr, usml,
      gmat)
    return out


def _kernel_core(feat0, feat1, feat2, feat3, feat4,
           cls_w, cls_b, cls_gn_g, cls_gn_b,
           reg_w, reg_b, reg_gn_g, reg_gn_b,
           pred_cls_w, pred_cls_b, pred_reg_w, pred_reg_b, scales):
    B = feat0.shape[0]
    bf = jnp.bfloat16

    def prep(f):
        f = jnp.transpose(f, (0, 2, 3, 1))
        return jnp.pad(f, ((0, 0), (1, 1), (1, 7), (0, 0)))

    xA = prep(feat0).reshape(B, 66 * 72, _C).astype(bf)

    p1 = prep(feat1)
    p2 = jnp.pad(jnp.transpose(feat2, (0, 2, 3, 1)),
                 ((0, 0), (1, 1), (1, 1), (0, 0)))
    p3 = jnp.pad(jnp.transpose(feat3, (0, 2, 3, 1)),
                 ((0, 0), (1, 9), (1, 1), (0, 0)))
    p4 = jnp.pad(jnp.transpose(feat4, (0, 2, 3, 1)),
                 ((0, 0), (1, 13), (1, 7), (0, 0)))
    band = jnp.concatenate([p2, p3, p4], axis=2)
    xB = jnp.concatenate([p1, band], axis=1).reshape(B, 52 * 40, _C).astype(bf)

    cw = cls_w.reshape(_SC, 9, _C, _C).astype(bf)
    rw = reg_w.reshape(_SC, 9, _C, _C).astype(bf)
    pcw = jnp.transpose(pred_cls_w,
                        (2, 1, 0, 3)).reshape(_C, 9 * _NC).astype(bf)
    prw = jnp.transpose(pred_reg_w,
                        (2, 1, 0, 3)).reshape(_C, 36).astype(bf)
    pcb2 = pred_cls_b.reshape(1, _NC)
    prb2 = pred_reg_b.reshape(1, 4)

    marrA = jnp.asarray(_MASK_A, bf)
    marrB = jnp.asarray(_MASK_B, bf)
    uvals = scales * jnp.asarray(_STRIDES, jnp.float32)
    uA = jnp.zeros((1, 8), jnp.float32).at[0, 0].set(uvals[0])
    uB = jnp.zeros((1, 8), jnp.float32).at[0, 0:4].set(uvals[1:5])

    oa = _run_canvas(xA, _CFG_A, marrA, uA, cw, cls_b, cls_gn_g,
                     cls_gn_b, rw, reg_b, reg_gn_g, reg_gn_b,
                     pcw, pcb2, prw, prb2)
    ob = _run_canvas(xB, _CFG_B, marrB, uB, cw, cls_b, cls_gn_g,
                     cls_gn_b, rw, reg_b, reg_gn_g, reg_gn_b,
                     pcw, pcb2, prw, prb2)

    ch = _NC + 4
    oa = oa.reshape(B, 66, 72, ch)[:, 1:65, 1:65].reshape(B, 4096, ch)
    ob4 = ob.reshape(B, 52, 40, ch)
    pieces = [oa]
    for (r0, r1, c0, c1, n) in ((1, 33, 1, 33, 1024), (35, 51, 1, 17, 256),
                                (35, 43, 19, 27, 64), (35, 39, 29, 33, 16)):
        pieces.append(ob4[:, r0:r1, c0:c1].reshape(B, n, ch))
    return jnp.concatenate(pieces, axis=1)


def kernel(feat0, feat1, feat2, feat3, feat4,
           cls_w, cls_b, cls_gn_g, cls_gn_b,
           reg_w, reg_b, reg_gn_g, reg_gn_b,
           pred_cls_w, pred_cls_b, pred_reg_w, pred_reg_b, scales):
    args = (feat0, feat1, feat2, feat3, feat4,
            cls_w, cls_b, cls_gn_g, cls_gn_b,
            reg_w, reg_b, reg_gn_g, reg_gn_b,
            pred_cls_w, pred_cls_b, pred_reg_w, pred_reg_b, scales)
    devs = jax.devices()
    B = feat0.shape[0]
    nd = min(len(devs), B)
    if nd < 2 or B % nd != 0:
        return _kernel_core(*args)
    try:
        from jax.experimental.shard_map import shard_map
    except ImportError:
        shard_map = jax.shard_map
    mesh = jax.sharding.Mesh(np.array(devs[:nd]), ("b",))
    P_ = jax.sharding.PartitionSpec
    batch = P_("b")
    rep = P_()
    in_specs = (batch,) * 5 + (rep,) * 13
    f = shard_map(_kernel_core, mesh=mesh, in_specs=in_specs,
                  out_specs=batch, check_rep=False)
    return f(*args)


# speedup vs baseline: 1.7193x; 1.7193x over previous
import functools

import numpy as np

import jax
import jax.numpy as jnp
from jax.experimental import pallas as pl
from jax.experimental.pallas import tpu as pltpu

_C = 256
_NC = 80
_SC = 4
_G = 32
_STRIDES = (8, 16, 32, 64, 128)


def _dot(a, b):
    return jax.lax.dot_general(a, b, (((1,), (0,)), ((), ())),
                               preferred_element_type=jnp.float32)


def _head_fn(x_ref, cw_ref, cb_ref, cg_ref, cbe_ref,
             rw_ref, rb_ref, rg_ref, rbe_ref,
             pcw_ref, pcb_ref, prw_ref, prb_ref,
             m_ref, u_ref, gm_ref,
             o_ref,
             t0c, t0p, t0m, t1c, t1p, t1m, *, cfg):
    Wc, P, LO, M = cfg["Wc"], cfg["P"], cfg["LO"], cfg["M"]
    ranges, segs, zeros = cfg["ranges"], cfg["segs"], cfg["zeros"]
    offs = [dy * Wc + dx for dy in (-1, 0, 1) for dx in (-1, 0, 1)]
    gmat = gm_ref[...]
    T0 = (t0c, t0p, t0m)
    T1 = (t1c, t1p, t1m)

    def load_x(dst3):
        c, p, m = dst3
        c[...] = x_ref[...]
        p[pl.ds(0, P - 1), :] = x_ref[pl.ds(1, P - 1), :]
        p[pl.ds(P - 1, 1), :] = jnp.zeros((1, _C), jnp.bfloat16)
        m[pl.ds(1, P - 1), :] = x_ref[pl.ds(0, P - 1), :]
        m[pl.ds(0, 1), :] = jnp.zeros((1, _C), jnp.bfloat16)

    def conv9(src3, w_ref, widx):
        acc = None
        for t in range(9):
            dy, dx = t // 3 - 1, t % 3 - 1
            buf = src3[0] if dx == 0 else (src3[1] if dx == 1 else src3[2])
            a = buf[pl.ds(LO + dy * Wc, M), :]
            d = _dot(a, w_ref[widx, t])
            acc = d if acc is None else acc + d
        return acc

    def store_triple(dst3, r0, n, zb):
        dst3[0][pl.ds(r0, n), :] = zb
        dst3[1][pl.ds(r0 - 1, n), :] = zb
        dst3[2][pl.ds(r0 + 1, n), :] = zb

    def zero_triple(dst3, z0, z1):
        dst3[0][pl.ds(z0, z1 - z0), :] = jnp.zeros((z1 - z0, _C), jnp.bfloat16)
        p0, p1 = max(z0 - 1, 0), max(z1 - 1, 0)
        if p1 > p0:
            dst3[1][pl.ds(p0, p1 - p0), :] = jnp.zeros((p1 - p0, _C),
                                                       jnp.bfloat16)
        m0, m1 = min(z0 + 1, P), min(z1 + 1, P)
        if m1 > m0:
            dst3[2][pl.ds(m0, m1 - m0), :] = jnp.zeros((m1 - m0, _C),
                                                       jnp.bfloat16)

    def gn_relu(y, g_ref, be_ref, i, dst3):
        stats = []
        for (r0, r1, mc, cnt) in ranges:
            ys = y[r0 - LO:r1 - LO]
            mm = m_ref[pl.ds(r0, r1 - r0), mc:mc + 1]
            ym = ys * mm
            stats.append(jnp.sum(ym, 0, keepdims=True))
            stats.append(jnp.sum(ym * ym, 0, keepdims=True))
        st = jnp.concatenate(stats, 0)
        hi = st.astype(jnp.bfloat16)
        lo_ = (st - hi.astype(jnp.float32)).astype(jnp.bfloat16)
        gs = _dot(hi, gmat) + _dot(lo_, gmat)
        g = g_ref[i:i + 1]
        be = be_ref[i:i + 1]
        scs, shs = [], []
        for r, (r0, r1, mc, cnt) in enumerate(ranges):
            mean = gs[2 * r:2 * r + 1] / cnt
            ex2 = gs[2 * r + 1:2 * r + 2] / cnt
            inv = jax.lax.rsqrt(ex2 - mean * mean + 1e-5)
            sc = g * inv
            scs.append(sc)
            shs.append(be - mean * sc)
        for (r0, r1, comps) in segs:
            ys = y[r0 - LO:r1 - LO]
            z = None
            for (mc, ridx) in comps:
                mm = m_ref[pl.ds(r0, r1 - r0), mc:mc + 1]
                zz = jnp.maximum(ys * scs[ridx] + shs[ridx], 0.0) * mm
                z = zz if z is None else z + zz
            store_triple(dst3, r0, r1 - r0, z.astype(jnp.bfloat16))
        for (z0, z1) in zeros:
            zero_triple(dst3, z0, z1)
        dst3[1][pl.ds(P - 1, 1), :] = jnp.zeros((1, _C), jnp.bfloat16)
        dst3[2][pl.ds(0, 1), :] = jnp.zeros((1, _C), jnp.bfloat16)

    def branch(first3, w_ref, g_ref, be_ref):
        load_x(first3)
        trips = [first3, T1 if first3 is T0 else T0]
        src = first3
        for i in range(_SC):
            dst3 = trips[(i + 1) % 2]
            y = conv9(src, w_ref, i)
            gn_relu(y, g_ref, be_ref, i, dst3)
            src = dst3
        return src

    src = branch(T0, cw_ref, cg_ref, cbe_ref)
    yc = pcb_ref[0:1]
    for kx in range(3):
        buf = (src[2], src[0], src[1])[kx]
        ptc = _dot(buf[...], pcw_ref[:, kx * 240:(kx + 1) * 240])
        for ky in range(3):
            r0 = LO + (ky - 1) * Wc
            yc = yc + ptc[r0:r0 + M, ky * _NC:(ky + 1) * _NC]

    src = branch(T1, rw_ref, rg_ref, rbe_ref)
    yr = prb_ref[0:1]
    for kx in range(3):
        buf = (src[2], src[0], src[1])[kx]
        prc = _dot(buf[...], prw_ref[:, kx * 12:(kx + 1) * 12])
        for ky in range(3):
            r0 = LO + (ky - 1) * Wc
            yr = yr + prc[r0:r0 + M, ky * 4:(ky + 1) * 4]
    parts = []
    for (r0, r1, comps) in segs:
        ys = yr[r0 - LO:r1 - LO]
        z = None
        for (mc, ridx) in comps:
            mm = m_ref[pl.ds(r0, r1 - r0), mc:mc + 1]
            uu = u_ref[0:1, ridx:ridx + 1]
            zz = jnp.maximum(ys * uu, 0.0) * mm
            z = zz if z is None else z + zz
        parts.append(z)
    o_ref[pl.ds(LO, M), :] = jnp.concatenate(
        [yc, jnp.concatenate(parts, axis=0)], axis=1)
    o_ref[pl.ds(0, LO), :] = jnp.zeros((LO, _NC + 4), jnp.float32)
    o_ref[pl.ds(P - LO, LO), :] = jnp.zeros((LO, _NC + 4), jnp.float32)


def _np_mask(rows, Wc, rects):
    m = np.zeros((rows, Wc, len(rects)), np.float32)
    for i, (r0, r1, c0, c1) in enumerate(rects):
        m[r0:r1, c0:c1, i] = 1.0
    return m.reshape(rows * Wc, len(rects))


_CFG_A = dict(Wc=72, P=4752, LO=72, M=4608,
              ranges=[(72, 4680, 0, 8 * 4096)],
              segs=[(72, 4680, [(0, 0)])],
              zeros=[(0, 72), (4680, 4752)])

_CFG_B = dict(Wc=40, P=2080, LO=40, M=2000,
              ranges=[(40, 1360, 0, 8 * 1024),
                      (1360, 2040, 1, 8 * 256),
                      (1360, 1760, 2, 8 * 64),
                      (1360, 1600, 3, 8 * 16)],
              segs=[(40, 1360, [(0, 0)]),
                    (1360, 2040, [(1, 1), (2, 2), (3, 3)])],
              zeros=[(0, 40), (2040, 2080)])

_MASK_A = _np_mask(66, 72, [(1, 65, 1, 65)])
_MASK_B = _np_mask(52, 40, [(1, 33, 1, 33),
                            (35, 51, 1, 17),
                            (35, 43, 19, 27),
                            (35, 39, 29, 33)])

_GMAT = (np.arange(_C)[:, None] // 8 == np.arange(_C)[None, :] // 8
         ).astype(np.float32)


def _run_canvas(x, cfg, marr, usml, cw, cb, cg, cbe, rw, rb, rg, rbe,
                pcw, pcb2, prw, prb2):
    B = x.shape[0]
    P = cfg["P"]
    ncol = marr.shape[1]
    gmat = jnp.asarray(_GMAT, jnp.bfloat16)
    kfn = functools.partial(_head_fn, cfg=cfg)
    full = lambda *shape: pl.BlockSpec(shape, lambda b: (0,) * len(shape))
    act = lambda: pltpu.VMEM((P, _C), jnp.bfloat16)
    out = pl.pallas_call(
        kfn,
        grid=(B,),
        in_specs=[
            pl.BlockSpec((None, P, _C), lambda b: (b, 0, 0)),
            full(_SC, 9, _C, _C), full(_SC, _C), full(_SC, _C), full(_SC, _C),
            full(_SC, 9, _C, _C), full(_SC, _C), full(_SC, _C), full(_SC, _C),
            full(_C, 9 * _NC), full(1, _NC), full(_C, 36), full(1, 4),
            full(P, ncol), full(1, 8), full(_C, _C),
        ],
        out_specs=pl.BlockSpec((None, P, _NC + 4), lambda b: (b, 0, 0)),
        out_shape=jax.ShapeDtypeStruct((B, P, _NC + 4), jnp.float32),
        scratch_shapes=[act(), act(), act(), act(), act(), act()],
        compiler_params=pltpu.CompilerParams(
            dimension_semantics=("parallel",)),
    )(x, cw, cb, cg, cbe, rw, rb, rg, rbe, pcw, pcb2, prw, prb2, marr, usml,
      gmat)
    return out


def _kernel_core(feat0, feat1, feat2, feat3, feat4,
           cls_w, cls_b, cls_gn_g, cls_gn_b,
           reg_w, reg_b, reg_gn_g, reg_gn_b,
           pred_cls_w, pred_cls_b, pred_reg_w, pred_reg_b, scales):
    B = feat0.shape[0]
    bf = jnp.bfloat16

    def prep(f):
        f = jnp.transpose(f, (0, 2, 3, 1))
        return jnp.pad(f, ((0, 0), (1, 1), (1, 7), (0, 0)))

    xA = prep(feat0).reshape(B, 66 * 72, _C).astype(bf)

    p1 = prep(feat1)
    p2 = jnp.pad(jnp.transpose(feat2, (0, 2, 3, 1)),
                 ((0, 0), (1, 1), (1, 1), (0, 0)))
    p3 = jnp.pad(jnp.transpose(feat3, (0, 2, 3, 1)),
                 ((0, 0), (1, 9), (1, 1), (0, 0)))
    p4 = jnp.pad(jnp.transpose(feat4, (0, 2, 3, 1)),
                 ((0, 0), (1, 13), (1, 7), (0, 0)))
    band = jnp.concatenate([p2, p3, p4], axis=2)
    xB = jnp.concatenate([p1, band], axis=1).reshape(B, 52 * 40, _C).astype(bf)

    cw = cls_w.reshape(_SC, 9, _C, _C).astype(bf)
    rw = reg_w.reshape(_SC, 9, _C, _C).astype(bf)
    pcw = jnp.transpose(pred_cls_w,
                        (2, 1, 0, 3)).reshape(_C, 9 * _NC).astype(bf)
    prw = jnp.transpose(pred_reg_w,
                        (2, 1, 0, 3)).reshape(_C, 36).astype(bf)
    pcb2 = pred_cls_b.reshape(1, _NC)
    prb2 = pred_reg_b.reshape(1, 4)

    marrA = jnp.asarray(_MASK_A, bf)
    marrB = jnp.asarray(_MASK_B, bf)
    uvals = scales * jnp.asarray(_STRIDES, jnp.float32)
    uA = jnp.zeros((1, 8), jnp.float32).at[0, 0].set(uvals[0])
    uB = jnp.zeros((1, 8), jnp.float32).at[0, 0:4].set(uvals[1:5])

    oa = _run_canvas(xA, _CFG_A, marrA, uA, cw, cls_b, cls_gn_g,
                     cls_gn_b, rw, reg_b, reg_gn_g, reg_gn_b,
                     pcw, pcb2, prw, prb2)
    ob = _run_canvas(xB, _CFG_B, marrB, uB, cw, cls_b, cls_gn_g,
                     cls_gn_b, rw, reg_b, reg_gn_g, reg_gn_b,
                     pcw, pcb2, prw, prb2)

    ch = _NC + 4
    oa = oa.reshape(B, 66, 72, ch)[:, 1:65, 1:65].reshape(B, 4096, ch)
    ob4 = ob.reshape(B, 52, 40, ch)
    pieces = [oa]
    for (r0, r1, c0, c1, n) in ((1, 33, 1, 33, 1024), (35, 51, 1, 17, 256),
                                (35, 43, 19, 27, 64), (35, 39, 29, 33, 16)):
        pieces.append(ob4[:, r0:r1, c0:c1].reshape(B, n, ch))
    return jnp.concatenate(pieces, axis=1)


def kernel(feat0, feat1, feat2, feat3, feat4,
           cls_w, cls_b, cls_gn_g, cls_gn_b,
           reg_w, reg_b, reg_gn_g, reg_gn_b,
           pred_cls_w, pred_cls_b, pred_reg_w, pred_reg_b, scales):
    return _kernel_core(feat0, feat1, feat2, feat3, feat4,
                        cls_w, cls_b, cls_gn_g, cls_gn_b,
                        reg_w, reg_b, reg_gn_g, reg_gn_b,
                        pred_cls_w, pred_cls_b, pred_reg_w, pred_reg_b,
                        scales)
